# baseline (speedup 1.0000x reference)
import jax
import jax.numpy as jnp
from jax import lax
from jax.experimental import pallas as pl
from jax.experimental.pallas import tpu as pltpu
from jax.experimental.pallas import tpu_sc as plsc

N_NODES = 10000
N_EDGES = 320000
D_FEAT = 128
HALF = D_FEAT // 2

NC = 2
NS = 16
CH = 128
NCHT = N_EDGES // CH
NCH = 160
ND = 4
BC = 8
NBS = 4
NBK = NCH // BC
CLAMP = NCHT - BC
DUMMY = N_NODES
N_RPAD = 10240
ROWS_PT = N_RPAD // NS
LAST_ROWS = N_NODES - 15 * ROWS_PT
FB = 40
NFB = ROWS_PT // FB
NFB_LAST = LAST_ROWS // FB
N_PAD = N_RPAD


def _sc_body(graph_r, x, eps16, out, acc, xsp, xb, ab, *ring):
  bufs = ring[:ND]
  sblk = ring[ND:ND + NBS]
  dblk = ring[ND + NBS:ND + 2 * NBS]
  gsem = ring[ND + 2 * NBS:2 * ND + 2 * NBS]
  ssem = ring[2 * ND + 2 * NBS:3 * ND + 2 * NBS]
  isem = ring[3 * ND + 2 * NBS:3 * ND + 3 * NBS]
  c = lax.axis_index("c")
  s = lax.axis_index("s")
  row0 = s * ROWS_PT
  off = 156 * s + jnp.minimum(s, 4)
  cnt = jnp.where(s < 4, 157, 156)

  def iblk_start(blk, bs):
    base = jnp.minimum(off + BC * blk, CLAMP)
    pltpu.make_async_copy(graph_r.at[0, pl.ds(base, BC)], sblk[bs],
                          isem[bs]).start()
    pltpu.make_async_copy(graph_r.at[1, pl.ds(base, BC)], dblk[bs],
                          isem[bs]).start()

  def iblk_wait(bs):
    pltpu.make_async_copy(graph_r.at[0, pl.ds(0, BC)], sblk[bs],
                          isem[bs]).wait()
    pltpu.make_async_copy(graph_r.at[1, pl.ds(0, BC)], dblk[bs],
                          isem[bs]).wait()

  def gather_copy(bs, p, k):
    return pltpu.make_async_copy(xsp.at[sblk[bs].at[p]], bufs[k], gsem[k])

  def scatter_wait(bs, p, k):
    pltpu.make_async_copy(bufs[k], acc.at[dblk[bs].at[p]], ssem[k]).wait()

  iblk_start(0, 0)
  iblk_start(1, 1)

  pltpu.sync_copy(eps16, ab.at[0, pl.ds(0, 16)])

  nrows = ROWS_PT if True else 0

  @pl.when(s < 15)
  def _():
    pltpu.make_async_copy(x.at[pl.ds(row0, ROWS_PT), pl.ds(c * HALF, HALF)],
                          xsp.at[pl.ds(row0, ROWS_PT)], gsem[3]).start()

  @pl.when(s == 15)
  def _():
    pltpu.make_async_copy(x.at[pl.ds(15 * ROWS_PT, LAST_ROWS),
                               pl.ds(c * HALF, HALF)],
                          xsp.at[pl.ds(15 * ROWS_PT, LAST_ROWS)],
                          gsem[3]).start()

  evec = ab[0, pl.ds(0, 16)]
  ez = evec[0]
  ev = 1.0 + evec

  @pl.when(jnp.logical_and(ez == 0.0, s < 15))
  def _():
    cp = pltpu.make_async_copy(
        x.at[pl.ds(row0, ROWS_PT), pl.ds(c * HALF, HALF)],
        acc.at[pl.ds(row0, ROWS_PT)], gsem[2])
    cp.start()
    cp.wait()

  @pl.when(jnp.logical_and(ez == 0.0, s == 15))
  def _():
    cp = pltpu.make_async_copy(
        x.at[pl.ds(15 * ROWS_PT, LAST_ROWS), pl.ds(c * HALF, HALF)],
        acc.at[pl.ds(15 * ROWS_PT, LAST_ROWS)], gsem[2])
    cp.start()
    cp.wait()

  @pl.when(s < 15)
  def _():
    pltpu.make_async_copy(x.at[pl.ds(row0, ROWS_PT), pl.ds(c * HALF, HALF)],
                          xsp.at[pl.ds(row0, ROWS_PT)], gsem[3]).wait()

  @pl.when(s == 15)
  def _():
    pltpu.make_async_copy(x.at[pl.ds(15 * ROWS_PT, LAST_ROWS),
                               pl.ds(c * HALF, HALF)],
                          xsp.at[pl.ds(15 * ROWS_PT, LAST_ROWS)],
                          gsem[3]).wait()

  @pl.when(ez != 0.0)
  def _():
    for b in range(NFB):
      r0 = row0 + b * FB
      buf = xb if b % 2 == 0 else ab
      pltpu.sync_copy(xsp.at[pl.ds(r0, FB)], buf)

      def scale_body(i, carry):
        brow = buf.at[i]
        for q in range(HALF // 16):
          sl = pl.ds(q * 16, 16)
          brow[sl] = ev * brow[sl]
        return carry

      lax.fori_loop(0, FB, scale_body, 0)
      pltpu.sync_copy(buf, acc.at[pl.ds(r0, FB)])
  plsc.subcore_barrier()

  iblk_wait(0)
  gather_copy(0, 0, 0).start()
  gather_copy(0, 1, 1).start()

  def patch_block19():
    base = jnp.minimum(off + BC * 19, CLAMP)
    dummy = DUMMY + lax.iota(jnp.int32, 16)
    for p in range(BC):
      bad = jnp.logical_or(base + p < off + 152, base + p >= off + cnt)

      @pl.when(bad)
      def _():
        drow = dblk[19 % NBS].at[p]
        for q in range(CH // 16):
          drow[pl.ds(q * 16, 16)] = dummy + 16 * q

  def front(q, i, guard, last):
    k = q % ND
    bs = (q // BC) % NBS
    p = q % BC
    gather_copy(bs, p, k).wait()
    pltpu.async_copy(bufs[k], acc.at[dblk[bs].at[p]], ssem[k], add=True)
    if guard:
      q2 = (q - 2) % 32
      scatter_wait((q2 // BC) % NBS, q2 % BC, q2 % ND)
    if p == 4:
      iblk_start(4 * i + q // BC + 2, (bs + 2) % NBS)
    if p == 5:
      iblk_wait((bs + 1) % NBS)
      if last and q == 21:
        patch_block19()
    q3 = (q + 2) % 32
    gather_copy((q3 // BC) % NBS, q3 % BC, q3 % ND).start()

  for q in range(32):
    front(q, 0, q >= 2, False)

  def edge_body(i, carry):
    for q in range(32):
      front(q, i, True, False)
    return carry

  lax.fori_loop(1, NCH // 32 - 1, edge_body, 0)

  for q in range(32):
    front(q, NCH // 32 - 1, True, True)

  for j in range(NCH - 2, NCH):
    q = j % 32
    scatter_wait((q // BC) % NBS, q % BC, q % ND)
  for j in range(NCH, NCH + 2):
    q = j % 32
    gather_copy((q // BC) % NBS, q % BC, q % ND).wait()
  iblk_wait((NBK + 1) % NBS)
  plsc.subcore_barrier()

  @pl.when(s < 15)
  def _():
    pltpu.sync_copy(acc.at[pl.ds(row0, ROWS_PT)],
                    out.at[pl.ds(row0, ROWS_PT), pl.ds(c * HALF, HALF)])

  @pl.when(s == 15)
  def _():
    pltpu.sync_copy(acc.at[pl.ds(15 * ROWS_PT, LAST_ROWS)],
                    out.at[pl.ds(15 * ROWS_PT, LAST_ROWS),
                           pl.ds(c * HALF, HALF)])


@jax.jit
def kernel(graph, x, eps):
  graph_r = graph.astype(jnp.int32).reshape(2, NCHT, CH)
  eps16 = jnp.broadcast_to(eps.astype(jnp.float32), (16,))

  fn = pl.kernel(
      _sc_body,
      out_type=jax.ShapeDtypeStruct((N_NODES, D_FEAT), jnp.float32),
      mesh=plsc.VectorSubcoreMesh(core_axis_name="c", subcore_axis_name="s"),
      compiler_params=pltpu.CompilerParams(use_tc_tiling_on_sc=False),
      scratch_types=[
          pltpu.VMEM_SHARED((N_PAD, HALF), jnp.float32),
          pltpu.VMEM_SHARED((N_PAD, HALF), jnp.float32),
          pltpu.VMEM((FB, HALF), jnp.float32),
          pltpu.VMEM((FB, HALF), jnp.float32),
      ] + [pltpu.VMEM((CH, HALF), jnp.float32)] * ND
        + [pltpu.VMEM((BC, CH), jnp.int32)] * NBS
        + [pltpu.VMEM((BC, CH), jnp.int32)] * NBS
        + [pltpu.SemaphoreType.DMA] * (2 * ND + NBS),
  )
  return fn(graph_r, x, eps16)


# speedup vs baseline: 1.0001x; 1.0001x over previous
import jax
import jax.numpy as jnp
from jax import lax
from jax.experimental import pallas as pl
from jax.experimental.pallas import tpu as pltpu
from jax.experimental.pallas import tpu_sc as plsc

N_NODES = 10000
N_EDGES = 320000
D_FEAT = 128
HALF = D_FEAT // 2

NC = 2
NS = 16
CH = 128
NCHT = N_EDGES // CH
NCH = 160
ND = 4
BC = 8
NBS = 4
NBK = NCH // BC
CLAMP = NCHT - BC
DUMMY = N_NODES
N_RPAD = 10240
ROWS_PT = N_RPAD // NS
LAST_ROWS = N_NODES - 15 * ROWS_PT
FB = 40
NFB = ROWS_PT // FB
NFB_LAST = LAST_ROWS // FB
N_PAD = N_RPAD


def _sc_body(graph_r, x, eps16, out, acc, xsp, xb, ab, *ring):
  bufs = ring[:ND]
  sblk = ring[ND:ND + NBS]
  dblk = ring[ND + NBS:ND + 2 * NBS]
  gsem = ring[ND + 2 * NBS:2 * ND + 2 * NBS]
  ssem = ring[2 * ND + 2 * NBS:3 * ND + 2 * NBS]
  isem = ring[3 * ND + 2 * NBS:3 * ND + 3 * NBS]
  c = lax.axis_index("c")
  s = lax.axis_index("s")
  row0 = s * ROWS_PT
  off = 156 * s + jnp.minimum(s, 4)
  cnt = jnp.where(s < 4, 157, 156)

  def iblk_start(blk, bs):
    base = jnp.minimum(off + BC * blk, CLAMP)
    pltpu.make_async_copy(graph_r.at[0, pl.ds(base, BC)], sblk[bs],
                          isem[bs]).start()
    pltpu.make_async_copy(graph_r.at[1, pl.ds(base, BC)], dblk[bs],
                          isem[bs]).start()

  def iblk_wait(bs):
    pltpu.make_async_copy(graph_r.at[0, pl.ds(0, BC)], sblk[bs],
                          isem[bs]).wait()
    pltpu.make_async_copy(graph_r.at[1, pl.ds(0, BC)], dblk[bs],
                          isem[bs]).wait()

  def gather_copy(bs, p, k):
    return pltpu.make_async_copy(xsp.at[sblk[bs].at[p]], bufs[k], gsem[k])

  def scatter_wait(bs, p, k):
    pltpu.make_async_copy(bufs[k], acc.at[dblk[bs].at[p]], ssem[k]).wait()

  iblk_start(0, 0)
  iblk_start(1, 1)

  pltpu.sync_copy(eps16, ab.at[0, pl.ds(0, 16)])

  @pl.when(s < 15)
  def _():
    pltpu.make_async_copy(x.at[pl.ds(row0, ROWS_PT), pl.ds(c * HALF, HALF)],
                          xsp.at[pl.ds(row0, ROWS_PT)], gsem[3]).start()

  @pl.when(s == 15)
  def _():
    pltpu.make_async_copy(x.at[pl.ds(15 * ROWS_PT, LAST_ROWS),
                               pl.ds(c * HALF, HALF)],
                          xsp.at[pl.ds(15 * ROWS_PT, LAST_ROWS)],
                          gsem[3]).start()

  evec = ab[0, pl.ds(0, 16)]
  ez = evec[0]
  ev = 1.0 + evec

  @pl.when(jnp.logical_and(ez == 0.0, s < 15))
  def _():
    cp = pltpu.make_async_copy(
        x.at[pl.ds(row0, ROWS_PT), pl.ds(c * HALF, HALF)],
        acc.at[pl.ds(row0, ROWS_PT)], gsem[2])
    cp.start()
    cp.wait()

  @pl.when(jnp.logical_and(ez == 0.0, s == 15))
  def _():
    cp = pltpu.make_async_copy(
        x.at[pl.ds(15 * ROWS_PT, LAST_ROWS), pl.ds(c * HALF, HALF)],
        acc.at[pl.ds(15 * ROWS_PT, LAST_ROWS)], gsem[2])
    cp.start()
    cp.wait()

  @pl.when(s < 15)
  def _():
    pltpu.make_async_copy(x.at[pl.ds(row0, ROWS_PT), pl.ds(c * HALF, HALF)],
                          xsp.at[pl.ds(row0, ROWS_PT)], gsem[3]).wait()

  @pl.when(s == 15)
  def _():
    pltpu.make_async_copy(x.at[pl.ds(15 * ROWS_PT, LAST_ROWS),
                               pl.ds(c * HALF, HALF)],
                          xsp.at[pl.ds(15 * ROWS_PT, LAST_ROWS)],
                          gsem[3]).wait()

  @pl.when(ez != 0.0)
  def _():
    for b in range(NFB):
      r0 = row0 + b * FB
      buf = xb if b % 2 == 0 else ab
      pltpu.sync_copy(xsp.at[pl.ds(r0, FB)], buf)

      def scale_body(i, carry):
        brow = buf.at[i]
        for q in range(HALF // 16):
          sl = pl.ds(q * 16, 16)
          brow[sl] = ev * brow[sl]
        return carry

      lax.fori_loop(0, FB, scale_body, 0)
      pltpu.sync_copy(buf, acc.at[pl.ds(r0, FB)])
  plsc.subcore_barrier()

  iblk_wait(0)
  gather_copy(0, 0, 0).start()
  gather_copy(0, 1, 1).start()

  def patch_block19():
    base = jnp.minimum(off + BC * 19, CLAMP)
    dummy = DUMMY + lax.iota(jnp.int32, 16)
    for p in range(BC):
      bad = jnp.logical_or(base + p < off + 152, base + p >= off + cnt)

      @pl.when(bad)
      def _():
        drow = dblk[19 % NBS].at[p]
        for q in range(CH // 16):
          drow[pl.ds(q * 16, 16)] = dummy + 16 * q

  def front(q, i, guard, last):
    k = q % ND
    bs = (q // BC) % NBS
    p = q % BC
    gather_copy(bs, p, k).wait()
    pltpu.async_copy(bufs[k], acc.at[dblk[bs].at[p]], ssem[k], add=True)
    if guard:
      q2 = (q - 2) % 32
      scatter_wait((q2 // BC) % NBS, q2 % BC, q2 % ND)
    if p == 4:
      iblk_start(4 * i + q // BC + 2, (bs + 2) % NBS)
    if p == 5:
      iblk_wait((bs + 1) % NBS)
      if last and q == 21:
        patch_block19()
    q3 = (q + 2) % 32
    gather_copy((q3 // BC) % NBS, q3 % BC, q3 % ND).start()

  for q in range(32):
    front(q, 0, q >= 2, False)

  def edge_body(i, carry):
    for q in range(32):
      front(q, i, True, False)
    return carry

  lax.fori_loop(1, NCH // 32 - 1, edge_body, 0)

  for q in range(32):
    front(q, NCH // 32 - 1, True, True)

  for j in range(NCH - 2, NCH):
    q = j % 32
    scatter_wait((q // BC) % NBS, q % BC, q % ND)
  for j in range(NCH, NCH + 2):
    q = j % 32
    gather_copy((q // BC) % NBS, q % BC, q % ND).wait()
  iblk_wait((NBK + 1) % NBS)
  plsc.subcore_barrier()

  @pl.when(s < 15)
  def _():
    pltpu.sync_copy(acc.at[pl.ds(row0, ROWS_PT)],
                    out.at[pl.ds(row0, ROWS_PT), pl.ds(c * HALF, HALF)])

  @pl.when(s == 15)
  def _():
    pltpu.sync_copy(acc.at[pl.ds(15 * ROWS_PT, LAST_ROWS)],
                    out.at[pl.ds(15 * ROWS_PT, LAST_ROWS),
                           pl.ds(c * HALF, HALF)])


@jax.jit
def kernel(graph, x, eps):
  graph_r = graph.astype(jnp.int32).reshape(2, NCHT, CH)
  eps16 = jnp.broadcast_to(eps.astype(jnp.float32), (16,))

  fn = pl.kernel(
      _sc_body,
      out_type=jax.ShapeDtypeStruct((N_NODES, D_FEAT), jnp.float32),
      mesh=plsc.VectorSubcoreMesh(core_axis_name="c", subcore_axis_name="s"),
      compiler_params=pltpu.CompilerParams(use_tc_tiling_on_sc=False),
      scratch_types=[
          pltpu.VMEM_SHARED((N_PAD, HALF), jnp.float32),
          pltpu.VMEM_SHARED((N_PAD, HALF), jnp.float32),
          pltpu.VMEM((FB, HALF), jnp.float32),
          pltpu.VMEM((FB, HALF), jnp.float32),
      ] + [pltpu.VMEM((CH, HALF), jnp.float32)] * ND
        + [pltpu.VMEM((BC, CH), jnp.int32)] * NBS
        + [pltpu.VMEM((BC, CH), jnp.int32)] * NBS
        + [pltpu.SemaphoreType.DMA] * (2 * ND + NBS),
  )
  return fn(graph_r, x, eps16)
